# baseline (speedup 1.0000x reference)
import functools

import jax
import jax.numpy as jnp
from jax import lax
from jax.experimental import pallas as pl
from jax.experimental.pallas import tpu as pltpu
from jax.experimental.pallas import tpu_sc as plsc

_NUM_WORKERS = 32
_CHUNK = 128
_NBUF = 4
_DIM = 32
_LANES = 16


@functools.partial(jax.jit, static_argnums=(2, 3))
def _emb_lookup(idx_t, weight, n_rows, n_tok):
    bs_per_w = n_rows // _NUM_WORKERS
    blocks_per_w = bs_per_w // _CHUNK
    n_chunks = n_tok * blocks_per_w
    n_groups = n_chunks // _NBUF
    n_bblocks = n_rows // _CHUNK
    d_tiles = _DIM // 8
    mesh = plsc.VectorSubcoreMesh(core_axis_name="c", subcore_axis_name="s")

    @functools.partial(
        pl.kernel,
        out_type=jax.ShapeDtypeStruct((n_tok, d_tiles, n_bblocks, 8, _CHUNK), jnp.float32),
        mesh=mesh,
        scratch_types=[
            pltpu.VMEM((n_tok, bs_per_w), jnp.int32),
            pltpu.VMEM((2, _NBUF, _CHUNK, _DIM), jnp.float32),
            pltpu.VMEM((2, _NBUF, d_tiles, 8, _CHUNK), jnp.float32),
            pltpu.SemaphoreType.DMA((2,)),
            pltpu.SemaphoreType.DMA((2,)),
        ],
        compiler_params=pltpu.CompilerParams(
            use_tc_tiling_on_sc=False, needs_layout_passes=False
        ),
    )
    def body(idx_hbm, table_hbm, out_hbm, idx_v, sbuf, tbuf, gsem, ssem):
        wid = lax.axis_index("s") * 2 + lax.axis_index("c")
        base = wid * bs_per_w
        pltpu.sync_copy(idx_hbm.at[:, pl.ds(base, bs_per_w)], idx_v)

        def chunk_pos(g, b):
            c = g * _NBUF + b
            t = lax.rem(c, n_tok)
            blk = c // n_tok
            return t, blk

        def gather_desc(g, b, sb):
            t, blk = chunk_pos(g, b)
            return pltpu.make_async_copy(
                table_hbm.at[idx_v.at[t, pl.ds(blk * _CHUNK, _CHUNK)]],
                sbuf.at[sb, b],
                gsem.at[sb],
            )

        def scatter_desc(g, b, sb):
            t, blk = chunk_pos(g, b)
            bglob = wid * blocks_per_w + blk
            return pltpu.make_async_copy(
                tbuf.at[sb, b],
                out_hbm.at[t, pl.ds(0, d_tiles), bglob],
                ssem.at[sb],
            )

        def launch_gathers(g, sb):
            for b in range(_NBUF):
                gather_desc(g, b, sb).start()

        def wait_gathers(g, sb):
            for b in range(_NBUF):
                gather_desc(g, b, sb).wait()

        def wait_scatters(g, sb):
            for b in range(_NBUF):
                scatter_desc(g, b, sb).wait()

        step = jnp.arange(_LANES, dtype=jnp.int32)

        def transpose_chunk(b, sb):
            gbuf = sbuf.at[sb, b]
            for d in range(_DIM):
                col = jnp.full((_LANES,), d, jnp.int32)
                dst = tbuf.at[sb, b, d // 8, d % 8]
                for k in range(_CHUNK // _LANES):
                    rows = step + (k * _LANES)
                    dst[pl.ds(k * _LANES, _LANES)] = plsc.load_gather(
                        gbuf, [rows, col]
                    )

        launch_gathers(0, 0)

        def group(g, carry):
            sb = lax.rem(g, 2)
            wait_gathers(g, sb)

            @pl.when(g >= 2)
            def _():
                wait_scatters(g - 2, sb)

            for b in range(_NBUF):
                transpose_chunk(b, sb)
                scatter_desc(g, b, sb).start()

            @pl.when(g + 1 < n_groups)
            def _():
                launch_gathers(g + 1, 1 - sb)

            return carry

        lax.fori_loop(0, n_groups, group, 0)
        wait_scatters(n_groups - 2, n_groups % 2)
        wait_scatters(n_groups - 1, (n_groups - 1) % 2)

    return body(idx_t, weight)


def kernel(token_ids, weight):
    n_rows, n_tok = token_ids.shape
    idx_t = jnp.maximum(token_ids.astype(jnp.int32), 0).T
    out5 = _emb_lookup(idx_t, weight, n_rows, n_tok)
    return out5.transpose(2, 4, 0, 1, 3).reshape(n_rows, n_tok, _DIM)
